# baseline (speedup 1.0000x reference)
import functools

import jax
import jax.numpy as jnp
from jax import lax
from jax.experimental import pallas as pl
from jax.experimental.pallas import tpu as pltpu
from jax.experimental.pallas import tpu_sc as plsc

VOCAB = 1000000
D_MODEL = 64
BATCH = 4096
SEQ = 200

_INFO = plsc.get_sparse_core_info()
_NC = _INFO.num_cores
_NS = _INFO.num_subcores
_NW = _NC * _NS
_B = BATCH * SEQ
_B_PER_W = _B // _NW
_CHUNK = 128
_N_CHUNKS = _B_PER_W // _CHUNK
_NBUF = 4
_LOOKAHEAD = _NBUF // 2

assert _B % _NW == 0 and _B_PER_W % _CHUNK == 0 and _N_CHUNKS > _NBUF


_NTC_FULL = VOCAB // 128
_TAIL = VOCAB - _NTC_FULL * 128
_T_STEPS = (_NTC_FULL + _NW - 1) // _NW


def _format_sc(table_t):
    mesh = plsc.VectorSubcoreMesh(core_axis_name="c", subcore_axis_name="s")

    @functools.partial(
        pl.kernel,
        mesh=mesh,
        out_type=jax.ShapeDtypeStruct((VOCAB, 128), jnp.float32),
        scratch_types=[
            pltpu.VMEM((64, 128), jnp.float32),
            pltpu.VMEM((64, 128), jnp.float32),
            pltpu.VMEM((128, 128), jnp.float32),
            pltpu.VMEM((128, 128), jnp.float32),
        ] + [pltpu.SemaphoreType.DMA for _ in range(4)],
        compiler_params=pltpu.CompilerParams(
            use_tc_tiling_on_sc=True, needs_layout_passes=False),
    )
    def k(tt_hbm, out_hbm, ib0, ib1, ob0, ob1, is0, is1, ws0, ws1):
        ibs, obs = (ib0, ib1), (ob0, ob1)
        isems, wsems = (is0, is1), (ws0, ws1)
        wid = lax.axis_index("s") * _NC + lax.axis_index("c")

        def blk(t):
            return jnp.minimum(t * _NW + wid, _NTC_FULL - 1)

        def start_in(t, b):
            pltpu.make_async_copy(
                tt_hbm.at[:, pl.ds(blk(t) * 128, 128)], ibs[b], isems[b]
            ).start()

        def wait_in(t, b):
            pltpu.make_async_copy(
                tt_hbm.at[:, pl.ds(blk(t) * 128, 128)], ibs[b], isems[b]
            ).wait()

        def start_out(t, b):
            pltpu.make_async_copy(
                obs[b], out_hbm.at[pl.ds(blk(t) * 128, 128)], wsems[b]
            ).start()

        def wait_out(t, b):
            pltpu.make_async_copy(
                obs[b], out_hbm.at[pl.ds(blk(t) * 128, 128)], wsems[b]
            ).wait()

        iota16 = lax.iota(jnp.int32, 16)

        def transpose_block(ib, ob, nrow):
            for r in range(nrow):
                rr = jnp.full((16,), r, jnp.int32)
                for q in range(4):
                    v = plsc.load_gather(ib, [iota16 + 16 * q, rr])
                    ob[r, pl.ds(16 * q, 16)] = v

        start_in(0, 0)
        start_in(1, 1)

        def body(t, carry):
            for b in range(2):
                tt = t * 2 + b
                wait_in(tt, b)

                @pl.when(tt >= 2)
                def _(tt=tt, b=b):
                    wait_out(tt - 2, b)

                transpose_block(ibs[b], obs[b], 128)
                start_out(tt, b)

                @pl.when(tt + 2 < _T_STEPS)
                def _(tt=tt, b=b):
                    start_in(tt + 2, b)
            return carry

        assert _T_STEPS % 2 == 1
        lax.fori_loop(0, (_T_STEPS - 1) // 2, body, 0)
        t_last = _T_STEPS - 1
        wait_in(t_last, 0)
        wait_out(t_last - 2, 0)
        transpose_block(ibs[0], obs[0], 128)
        start_out(t_last, 0)
        wait_out(t_last - 1, 1)
        wait_out(t_last, 0)

    return k(table_t)


def _gather_sc(table, idx_flat):
    mesh = plsc.VectorSubcoreMesh(core_axis_name="c", subcore_axis_name="s")

    @functools.partial(
        pl.kernel,
        mesh=mesh,
        out_type=jax.ShapeDtypeStruct((_B, 128), jnp.float32),
        scratch_types=[
            pltpu.VMEM((_B_PER_W,), jnp.int32),
        ] + [pltpu.VMEM((_CHUNK, 128), jnp.float32) for _ in range(_NBUF)]
          + [pltpu.SemaphoreType.DMA for _ in range(2 * _NBUF)],
        compiler_params=pltpu.CompilerParams(use_tc_tiling_on_sc=True),
    )
    def k(table_hbm, idx_hbm, out_hbm, idx_all, *bufs_and_sems):
        bufs = bufs_and_sems[:_NBUF]
        gsems = bufs_and_sems[_NBUF:2 * _NBUF]
        wsems = bufs_and_sems[2 * _NBUF:]
        wid = lax.axis_index("s") * _NC + lax.axis_index("c")
        base = wid * _B_PER_W

        pltpu.sync_copy(idx_hbm.at[pl.ds(base, _B_PER_W)], idx_all)

        def start_gather(chunk, b):
            pltpu.make_async_copy(
                table_hbm.at[idx_all.at[pl.ds(chunk * _CHUNK, _CHUNK)]],
                bufs[b], gsems[b]).start()

        def wait_gather(chunk, b):
            pltpu.make_async_copy(
                table_hbm.at[idx_all.at[pl.ds(chunk * _CHUNK, _CHUNK)]],
                bufs[b], gsems[b]).wait()

        def start_write(chunk, b):
            pltpu.make_async_copy(
                bufs[b], out_hbm.at[pl.ds(base + chunk * _CHUNK, _CHUNK)],
                wsems[b]).start()

        def wait_write(chunk, b):
            pltpu.make_async_copy(
                bufs[b], out_hbm.at[pl.ds(base + chunk * _CHUNK, _CHUNK)],
                wsems[b]).wait()

        for j in range(_LOOKAHEAD):
            start_gather(j, j % _NBUF)

        def body(g, carry):
            for bb in range(_NBUF):
                j = g * _NBUF + bb
                wait_gather(j, bb)
                start_write(j, bb)
                t = j + _LOOKAHEAD
                bt = (bb + _LOOKAHEAD) % _NBUF

                @pl.when(t >= _NBUF)
                def _(t=t, bt=bt):
                    wait_write(t - _NBUF, bt)

                @pl.when(t < _N_CHUNKS)
                def _(t=t, bt=bt):
                    start_gather(t, bt)
            return carry

        lax.fori_loop(0, _N_CHUNKS // _NBUF, body, 0)

        for j in range(_N_CHUNKS - _LOOKAHEAD, _N_CHUNKS):
            wait_write(j, j % _NBUF)

    return k(table, idx_flat)


def kernel(input_ids, shared_weight):
    idx_flat = input_ids.reshape(-1).astype(jnp.int32)
    table128 = _format_sc(shared_weight.T)
    tail = jnp.pad(shared_weight[_NTC_FULL * 128:], ((0, 0), (0, 128 - D_MODEL)))
    table128 = jax.lax.dynamic_update_slice(table128, tail, (_NTC_FULL * 128, 0))
    out = _gather_sc(table128, idx_flat)[:, :D_MODEL]
    return out.reshape(*input_ids.shape, D_MODEL)


# speedup vs baseline: 2.1583x; 2.1583x over previous
import functools

import jax
import jax.numpy as jnp
from jax import lax
from jax.experimental import pallas as pl
from jax.experimental.pallas import tpu as pltpu
from jax.experimental.pallas import tpu_sc as plsc

VOCAB = 1000000
D_MODEL = 64
BATCH = 4096
SEQ = 200

_INFO = plsc.get_sparse_core_info()
_NC = _INFO.num_cores
_NS = _INFO.num_subcores
_NW = _NC * _NS
_B = BATCH * SEQ
_B_PER_W = _B // _NW
_CHUNK = 200
_N_CHUNKS = _B_PER_W // _CHUNK
_NBUF = 4
_LOOKAHEAD = _NBUF // 2

assert _B % _NW == 0 and _B_PER_W % _CHUNK == 0 and _N_CHUNKS % _NBUF == 0


def _gather_sc(table, idx_flat):
    mesh = plsc.VectorSubcoreMesh(core_axis_name="c", subcore_axis_name="s")

    @functools.partial(
        pl.kernel,
        mesh=mesh,
        out_type=jax.ShapeDtypeStruct((_B, 128), jnp.float32),
        scratch_types=[
            pltpu.VMEM((_B_PER_W,), jnp.int32),
        ] + [pltpu.VMEM((_CHUNK, 128), jnp.float32) for _ in range(_NBUF)]
          + [pltpu.SemaphoreType.DMA for _ in range(2 * _NBUF)],
        compiler_params=pltpu.CompilerParams(use_tc_tiling_on_sc=True),
    )
    def k(table_hbm, idx_hbm, out_hbm, idx_all, *bufs_and_sems):
        bufs = bufs_and_sems[:_NBUF]
        gsems = bufs_and_sems[_NBUF:2 * _NBUF]
        wsems = bufs_and_sems[2 * _NBUF:]
        wid = lax.axis_index("s") * _NC + lax.axis_index("c")
        base = wid * _B_PER_W

        pltpu.sync_copy(idx_hbm.at[pl.ds(base, _B_PER_W)], idx_all)

        def start_gather(chunk, b):
            pltpu.make_async_copy(
                table_hbm.at[idx_all.at[pl.ds(chunk * _CHUNK, _CHUNK)]],
                bufs[b], gsems[b]).start()

        def wait_gather(chunk, b):
            pltpu.make_async_copy(
                table_hbm.at[idx_all.at[pl.ds(chunk * _CHUNK, _CHUNK)]],
                bufs[b], gsems[b]).wait()

        def start_write(chunk, b):
            pltpu.make_async_copy(
                bufs[b], out_hbm.at[pl.ds(base + chunk * _CHUNK, _CHUNK)],
                wsems[b]).start()

        def wait_write(chunk, b):
            pltpu.make_async_copy(
                bufs[b], out_hbm.at[pl.ds(base + chunk * _CHUNK, _CHUNK)],
                wsems[b]).wait()

        for j in range(_LOOKAHEAD):
            start_gather(j, j % _NBUF)

        def body(g, carry):
            for bb in range(_NBUF):
                j = g * _NBUF + bb
                wait_gather(j, bb)
                start_write(j, bb)
                t = j + _LOOKAHEAD
                bt = (bb + _LOOKAHEAD) % _NBUF

                @pl.when(t >= _NBUF)
                def _(t=t, bt=bt):
                    wait_write(t - _NBUF, bt)

                @pl.when(t < _N_CHUNKS)
                def _(t=t, bt=bt):
                    start_gather(t, bt)
            return carry

        lax.fori_loop(0, _N_CHUNKS // _NBUF, body, 0)

        for j in range(_N_CHUNKS - _LOOKAHEAD, _N_CHUNKS):
            wait_write(j, j % _NBUF)

    return k(table, idx_flat)


def kernel(input_ids, shared_weight):
    idx_flat = input_ids.reshape(-1).astype(jnp.int32)
    table128 = jnp.pad(shared_weight, ((0, 0), (0, 128 - D_MODEL)))
    out = _gather_sc(table128, idx_flat)[:, :D_MODEL]
    return out.reshape(*input_ids.shape, D_MODEL)
